# Initial kernel scaffold; baseline (speedup 1.0000x reference)
import jax
import jax.numpy as jnp
from jax.experimental import pallas as pl


def kernel(x, edge_index, batch, W1_l, b1_l, W1_r, W2_l, b2_l, W2_r, W_fc, b_fc):
    raise NotImplementedError("write your pallas kernel here")


# speedup vs baseline: 8.5380x; 8.5380x over previous
import functools

import jax
import jax.numpy as jnp
from jax import lax
from jax.experimental import pallas as pl
from jax.experimental.pallas import tpu as pltpu
from jax.experimental.pallas import tpu_sc as plsc

N = 10000
E = 320000
D = 128
G = 16

NC = 2
NS = 16
NW = NC * NS
CHUNK = 128
EPW = E // NW
NCHUNK = -(-EPW // CHUNK)
EPW_PAD = NCHUNK * CHUNK
E_PAD = EPW_PAD * NW
ACC_ROWS = 10240
RPT = ACC_ROWS // NS
ORT = N // NS
DEG_W = 4

BS = 400
NB = N // BS


def _sc_agg_body(with_deg, *refs):
    if with_deg:
        (x_hbm, src_hbm, dst_hbm, zrow_hbm, zdeg_hbm, ones_hbm,
         part_hbm, degp_hbm,
         src_v, dst_v, rows_v, ones_v, acc_sh, deg_sh, sem) = refs
    else:
        (x_hbm, src_hbm, dst_hbm, zrow_hbm,
         part_hbm,
         src_v, dst_v, rows_v, acc_sh, sem) = refs

    c = lax.axis_index("c")
    s = lax.axis_index("s")
    wid = c * NS + s

    pltpu.sync_copy(zrow_hbm, acc_sh.at[pl.ds(s * RPT, RPT)])
    if with_deg:
        pltpu.sync_copy(zdeg_hbm, deg_sh.at[pl.ds(s * RPT, RPT)])
        pltpu.sync_copy(ones_hbm, ones_v)

    pltpu.sync_copy(src_hbm.at[wid], src_v)
    pltpu.sync_copy(dst_hbm.at[wid], dst_v)

    plsc.subcore_barrier()

    def chunk_body(j, carry):
        pltpu.async_copy(x_hbm.at[src_v.at[j]], rows_v, sem).wait()
        pltpu.sync_copy(rows_v, acc_sh.at[dst_v.at[j]], add=True)
        if with_deg:
            pltpu.sync_copy(ones_v, deg_sh.at[dst_v.at[j]], add=True)
        return carry

    lax.fori_loop(0, NCHUNK, chunk_body, 0)

    plsc.subcore_barrier()

    pltpu.sync_copy(acc_sh.at[pl.ds(s * RPT, RPT)],
                    part_hbm.at[c, pl.ds(s * RPT, RPT)])
    if with_deg:
        pltpu.sync_copy(deg_sh.at[pl.ds(s * RPT, RPT)],
                        degp_hbm.at[c, pl.ds(s * RPT, RPT)])


def _make_sc_agg(with_deg):
    mesh = plsc.VectorSubcoreMesh(core_axis_name="c", subcore_axis_name="s")
    part_t = jax.ShapeDtypeStruct((NC, ACC_ROWS, D), jnp.float32)
    out_type = [part_t] if with_deg else part_t
    scratch = [
        pltpu.VMEM((NCHUNK, CHUNK), jnp.int32),
        pltpu.VMEM((NCHUNK, CHUNK), jnp.int32),
        pltpu.VMEM((CHUNK, D), jnp.float32),
    ]
    if with_deg:
        out_type.append(jax.ShapeDtypeStruct((NC, ACC_ROWS), jnp.float32))
        scratch.append(pltpu.VMEM((CHUNK,), jnp.float32))
    scratch.append(pltpu.VMEM_SHARED((ACC_ROWS, D), jnp.float32))
    if with_deg:
        scratch.append(pltpu.VMEM_SHARED((ACC_ROWS,), jnp.float32))
    scratch.append(pltpu.SemaphoreType.DMA)
    return pl.kernel(
        functools.partial(_sc_agg_body, with_deg),
        out_type=out_type,
        mesh=mesh,
        scratch_types=scratch,
    )


_sc_agg_deg = _make_sc_agg(True)
_sc_agg = _make_sc_agg(False)


def _rowmat(a, b):
    return lax.dot_general(a, b, (((1,), (1,)), ((), ())),
                           preferred_element_type=jnp.float32)


def _tc1_body(p0, p1, dall, x, w_l, b_l, w_r, h_ref):
    d = jnp.sum(dall[...], axis=1, keepdims=True)
    agg = (p0[...] + p1[...]) / jnp.maximum(d, 1.0)
    z = _rowmat(agg, w_l[...]) + _rowmat(x[...], w_r[...]) + b_l[...]
    h_ref[...] = jnp.maximum(z, 0.0)


def _tc2_body(q0, q1, dall, h1, b3, w_l, b_l, w_r, wfc, bfc,
              out_ref, sums, cnts):
    i = pl.program_id(0)
    d = jnp.sum(dall[...], axis=1, keepdims=True)
    agg = (q0[...] + q1[...]) / jnp.maximum(d, 1.0)
    z = _rowmat(agg, w_l[...]) + _rowmat(h1[...], w_r[...]) + b_l[...]
    h2 = jnp.maximum(z, 0.0)

    b = b3[...][0, 0, :]
    m = (lax.broadcasted_iota(jnp.int32, (G, BS), 0)
         == b[None, :]).astype(jnp.float32)
    ps = lax.dot_general(m, h2, (((1,), (0,)), ((), ())),
                         precision=lax.Precision.HIGHEST,
                         preferred_element_type=jnp.float32)
    pc = lax.dot_general(m, jnp.ones((BS, D), jnp.float32),
                         (((1,), (0,)), ((), ())),
                         preferred_element_type=jnp.float32)

    @pl.when(i == 0)
    def _():
        sums[...] = ps
        cnts[...] = pc

    @pl.when(i > 0)
    def _():
        sums[...] += ps
        cnts[...] += pc

    @pl.when(i == NB - 1)
    def _():
        pooled = sums[...] / jnp.maximum(cnts[...], 1.0)
        out_ref[...] = jnp.sum(pooled * wfc[...], axis=1) + bfc[0]


_row_spec = pl.BlockSpec((BS, D), lambda i: (i, 0))
_deg_spec = pl.BlockSpec((BS, NC), lambda i: (i, 0))
_w_spec = pl.BlockSpec((D, D), lambda i: (0, 0))
_b_spec = pl.BlockSpec((D,), lambda i: (0,))

_tc1 = pl.pallas_call(
    _tc1_body,
    grid=(NB,),
    in_specs=[_row_spec, _row_spec, _deg_spec, _row_spec,
              _w_spec, _b_spec, _w_spec],
    out_specs=_row_spec,
    out_shape=jax.ShapeDtypeStruct((N, D), jnp.float32),
)

_tc2 = pl.pallas_call(
    _tc2_body,
    grid=(NB,),
    in_specs=[_row_spec, _row_spec, _deg_spec, _row_spec,
              pl.BlockSpec((1, 1, BS), lambda i: (i, 0, 0)),
              _w_spec, _b_spec, _w_spec,
              pl.BlockSpec((1, D), lambda i: (0, 0)),
              pl.BlockSpec(memory_space=pltpu.MemorySpace.SMEM)],
    out_specs=pl.BlockSpec((G,), lambda i: (0,)),
    out_shape=jax.ShapeDtypeStruct((G,), jnp.float32),
    scratch_shapes=[pltpu.VMEM((G, D), jnp.float32),
                    pltpu.VMEM((G, D), jnp.float32)],
)


def kernel(x, edge_index, batch, W1_l, b1_l, W1_r, W2_l, b2_l, W2_r,
           W_fc, b_fc):
    src = edge_index[0]
    dst = edge_index[1]
    pad = E_PAD - E
    padi = jnp.arange(pad, dtype=jnp.int32)
    src_p = jnp.concatenate([src, padi % N]).reshape(NW, NCHUNK, CHUNK)
    dst_p = jnp.concatenate([dst, N + padi % (ACC_ROWS - N)]
                            ).reshape(NW, NCHUNK, CHUNK)
    zrow = jnp.zeros((RPT, D), jnp.float32)
    zdeg = jnp.zeros((RPT,), jnp.float32)
    ones = jnp.ones((CHUNK,), jnp.float32)

    part1, degp = _sc_agg_deg(x, src_p, dst_p, zrow, zdeg, ones)
    part1, degT = part1[:, :N], degp[:, :N].T
    h1 = _tc1(part1[0], part1[1], degT, x, W1_l, b1_l, W1_r)
    part2 = _sc_agg(h1, src_p, dst_p, zrow)[:, :N]
    b3 = batch.reshape(NB, 1, BS)
    out = _tc2(part2[0], part2[1], degT, h1, b3,
               W2_l, b2_l, W2_r, W_fc, b_fc)
    return out


# speedup vs baseline: 12.4196x; 1.4546x over previous
import functools

import jax
import jax.numpy as jnp
from jax import lax
from jax.experimental import pallas as pl
from jax.experimental.pallas import tpu as pltpu
from jax.experimental.pallas import tpu_sc as plsc

N = 10000
E = 320000
D = 128
G = 16

NC = 2
NS = 16
NW = NC * NS
CHUNK = 128
EPW = E // NW
NCHUNK = -(-EPW // CHUNK)
EPW_PAD = NCHUNK * CHUNK
E_PAD = EPW_PAD * NW
ACC_ROWS = 10240
RPT = ACC_ROWS // NS
ORT = N // NS
DEG_W = 4

BS = 400
NB = N // BS


def _sc_agg_body(with_deg, *refs):
    if with_deg:
        (x_hbm, eidx_hbm, zrow_hbm, zdeg_hbm, ones_hbm,
         part_hbm, degp_hbm,
         idx_v, rows_v, ones_v, acc_sh, deg_sh, sem_g, sem_i) = refs
    else:
        (x_hbm, eidx_hbm, zrow_hbm,
         part_hbm,
         idx_v, rows_v, acc_sh, sem_g, sem_i) = refs

    c = lax.axis_index("c")
    s = lax.axis_index("s")
    wid = c * NS + s

    pltpu.sync_copy(zrow_hbm, acc_sh.at[pl.ds(s * RPT, RPT)])
    if with_deg:
        pltpu.sync_copy(zdeg_hbm, deg_sh.at[pl.ds(s * RPT, RPT)])
        pltpu.sync_copy(ones_hbm, ones_v)

    pltpu.async_copy(eidx_hbm.at[wid, 0], idx_v.at[0], sem_i)
    pltpu.async_copy(eidx_hbm.at[wid, 1], idx_v.at[1], sem_i)

    plsc.subcore_barrier()

    pltpu.make_async_copy(eidx_hbm.at[wid, 0], idx_v.at[0], sem_i).wait()
    pltpu.async_copy(x_hbm.at[idx_v.at[0, 0]], rows_v.at[0], sem_g)

    def chunk_body(j, carry):
        buf = lax.rem(j, 2)
        slot = lax.rem(j, 4)

        @pl.when(j + 1 < NCHUNK)
        def _():
            nslot = lax.rem(j + 1, 4)
            pltpu.make_async_copy(eidx_hbm.at[wid, j + 1], idx_v.at[nslot],
                                  sem_i).wait()
            pltpu.async_copy(x_hbm.at[idx_v.at[nslot, 0]],
                             rows_v.at[1 - buf], sem_g)

            @pl.when(j + 2 < NCHUNK)
            def _():
                pltpu.async_copy(eidx_hbm.at[wid, j + 2],
                                 idx_v.at[lax.rem(j + 2, 4)], sem_i)

        pltpu.make_async_copy(x_hbm.at[idx_v.at[slot, 0]], rows_v.at[buf],
                              sem_g).wait()
        pltpu.sync_copy(rows_v.at[buf], acc_sh.at[idx_v.at[slot, 1]],
                        add=True)
        if with_deg:
            pltpu.sync_copy(ones_v, deg_sh.at[idx_v.at[slot, 1]], add=True)
        return carry

    lax.fori_loop(0, NCHUNK, chunk_body, 0)

    plsc.subcore_barrier()

    pltpu.sync_copy(acc_sh.at[pl.ds(s * RPT, RPT)],
                    part_hbm.at[c, pl.ds(s * RPT, RPT)])
    if with_deg:
        pltpu.sync_copy(deg_sh.at[pl.ds(s * RPT, RPT)],
                        degp_hbm.at[c, pl.ds(s * RPT, RPT)])


def _make_sc_agg(with_deg):
    mesh = plsc.VectorSubcoreMesh(core_axis_name="c", subcore_axis_name="s")
    part_t = jax.ShapeDtypeStruct((NC, ACC_ROWS, D), jnp.float32)
    out_type = [part_t] if with_deg else part_t
    scratch = [
        pltpu.VMEM((4, 2, CHUNK), jnp.int32),
        pltpu.VMEM((2, CHUNK, D), jnp.float32),
    ]
    if with_deg:
        out_type.append(jax.ShapeDtypeStruct((NC, ACC_ROWS), jnp.float32))
        scratch.append(pltpu.VMEM((CHUNK,), jnp.float32))
    scratch.append(pltpu.VMEM_SHARED((ACC_ROWS, D), jnp.float32))
    if with_deg:
        scratch.append(pltpu.VMEM_SHARED((ACC_ROWS,), jnp.float32))
    scratch.append(pltpu.SemaphoreType.DMA)
    scratch.append(pltpu.SemaphoreType.DMA)
    return pl.kernel(
        functools.partial(_sc_agg_body, with_deg),
        out_type=out_type,
        mesh=mesh,
        scratch_types=scratch,
    )


_sc_agg_deg = _make_sc_agg(True)
_sc_agg = _make_sc_agg(False)


def _rowmat(a, b):
    return lax.dot_general(a, b, (((1,), (1,)), ((), ())),
                           preferred_element_type=jnp.float32)


def _tc1_body(p0, p1, dall, x, w_l, b_l, w_r, h_ref):
    d = jnp.sum(dall[...], axis=1, keepdims=True)
    agg = (p0[...] + p1[...]) / jnp.maximum(d, 1.0)
    z = _rowmat(agg, w_l[...]) + _rowmat(x[...], w_r[...]) + b_l[...]
    h_ref[...] = jnp.maximum(z, 0.0)


def _tc2_body(q0, q1, dall, h1, b3, w_l, b_l, w_r, wfc, bfc,
              out_ref, sums, cnts):
    i = pl.program_id(0)
    d = jnp.sum(dall[...], axis=1, keepdims=True)
    agg = (q0[...] + q1[...]) / jnp.maximum(d, 1.0)
    z = _rowmat(agg, w_l[...]) + _rowmat(h1[...], w_r[...]) + b_l[...]
    h2 = jnp.maximum(z, 0.0)

    b = b3[...][0, 0, :]
    m = (lax.broadcasted_iota(jnp.int32, (G, BS), 0)
         == b[None, :]).astype(jnp.float32)
    ps = lax.dot_general(m, h2, (((1,), (0,)), ((), ())),
                         precision=lax.Precision.HIGHEST,
                         preferred_element_type=jnp.float32)
    pc = lax.dot_general(m, jnp.ones((BS, D), jnp.float32),
                         (((1,), (0,)), ((), ())),
                         preferred_element_type=jnp.float32)

    @pl.when(i == 0)
    def _():
        sums[...] = ps
        cnts[...] = pc

    @pl.when(i > 0)
    def _():
        sums[...] += ps
        cnts[...] += pc

    @pl.when(i == NB - 1)
    def _():
        pooled = sums[...] / jnp.maximum(cnts[...], 1.0)
        out_ref[...] = jnp.sum(pooled * wfc[...], axis=1) + bfc[0]


_row_spec = pl.BlockSpec((BS, D), lambda i: (i, 0))
_deg_spec = pl.BlockSpec((BS, NC), lambda i: (i, 0))
_w_spec = pl.BlockSpec((D, D), lambda i: (0, 0))
_b_spec = pl.BlockSpec((D,), lambda i: (0,))

_tc1 = pl.pallas_call(
    _tc1_body,
    grid=(NB,),
    in_specs=[_row_spec, _row_spec, _deg_spec, _row_spec,
              _w_spec, _b_spec, _w_spec],
    out_specs=_row_spec,
    out_shape=jax.ShapeDtypeStruct((N, D), jnp.float32),
)

_tc2 = pl.pallas_call(
    _tc2_body,
    grid=(NB,),
    in_specs=[_row_spec, _row_spec, _deg_spec, _row_spec,
              pl.BlockSpec((1, 1, BS), lambda i: (i, 0, 0)),
              _w_spec, _b_spec, _w_spec,
              pl.BlockSpec((1, D), lambda i: (0, 0)),
              pl.BlockSpec(memory_space=pltpu.MemorySpace.SMEM)],
    out_specs=pl.BlockSpec((G,), lambda i: (0,)),
    out_shape=jax.ShapeDtypeStruct((G,), jnp.float32),
    scratch_shapes=[pltpu.VMEM((G, D), jnp.float32),
                    pltpu.VMEM((G, D), jnp.float32)],
)


def kernel(x, edge_index, batch, W1_l, b1_l, W1_r, W2_l, b2_l, W2_r,
           W_fc, b_fc):
    src = edge_index[0]
    dst = edge_index[1]
    pad = E_PAD - E
    padi = jnp.arange(pad, dtype=jnp.int32)
    src_p = jnp.concatenate([src, padi % N]).reshape(NW, NCHUNK, 1, CHUNK)
    dst_p = jnp.concatenate([dst, N + padi % (ACC_ROWS - N)]
                            ).reshape(NW, NCHUNK, 1, CHUNK)
    eidx = jnp.concatenate([src_p, dst_p], axis=2)
    zrow = jnp.zeros((RPT, D), jnp.float32)
    zdeg = jnp.zeros((RPT,), jnp.float32)
    ones = jnp.ones((CHUNK,), jnp.float32)

    part1, degp = _sc_agg_deg(x, eidx, zrow, zdeg, ones)
    part1, degT = part1[:, :N], degp[:, :N].T
    h1 = _tc1(part1[0], part1[1], degT, x, W1_l, b1_l, W1_r)
    part2 = _sc_agg(h1, eidx, zrow)[:, :N]
    b3 = batch.reshape(NB, 1, BS)
    out = _tc2(part2[0], part2[1], degT, h1, b3,
               W2_l, b2_l, W2_r, W_fc, b_fc)
    return out


# speedup vs baseline: 12.9765x; 1.0448x over previous
import functools

import jax
import jax.numpy as jnp
from jax import lax
from jax.experimental import pallas as pl
from jax.experimental.pallas import tpu as pltpu
from jax.experimental.pallas import tpu_sc as plsc

N = 10000
E = 320000
D = 128
G = 16

NC = 2
NS = 16
NW = NC * NS
CHUNK = 128
EPW = E // NW
NCHUNK = -(-EPW // CHUNK)
EPW_PAD = NCHUNK * CHUNK
E_PAD = EPW_PAD * NW
ACC_ROWS = 10240
RPT = ACC_ROWS // NS
ORT = N // NS
DEG_W = 4

BS = 400
NB = N // BS


def _sc_agg_body(with_deg, *refs):
    if with_deg:
        (x_hbm, eidx_hbm, zrow_hbm, zdeg_hbm, ones_hbm,
         part_hbm, degp_hbm,
         idx_v, rows_v, ones_v, acc_sh, deg_sh, sem_g, sem_i) = refs
    else:
        (x_hbm, eidx_hbm, zrow_hbm,
         part_hbm,
         idx_v, rows_v, acc_sh, sem_g, sem_i) = refs

    c = lax.axis_index("c")
    s = lax.axis_index("s")
    wid = c * NS + s

    pltpu.sync_copy(zrow_hbm, acc_sh.at[pl.ds(s * RPT, RPT)])
    if with_deg:
        pltpu.sync_copy(zdeg_hbm, deg_sh.at[pl.ds(s * RPT, RPT)])
        pltpu.sync_copy(ones_hbm, ones_v)

    pltpu.async_copy(eidx_hbm.at[wid, 0], idx_v.at[0], sem_i)
    pltpu.async_copy(eidx_hbm.at[wid, 1], idx_v.at[1], sem_i)

    plsc.subcore_barrier()

    pltpu.make_async_copy(eidx_hbm.at[wid, 0], idx_v.at[0], sem_i).wait()
    pltpu.async_copy(x_hbm.at[idx_v.at[0, 0]], rows_v.at[0], sem_g)

    def chunk_body(j, carry):
        buf = lax.rem(j, 2)
        slot = lax.rem(j, 4)

        @pl.when(j + 1 < NCHUNK)
        def _():
            nslot = lax.rem(j + 1, 4)
            pltpu.make_async_copy(eidx_hbm.at[wid, j + 1], idx_v.at[nslot],
                                  sem_i).wait()
            pltpu.async_copy(x_hbm.at[idx_v.at[nslot, 0]],
                             rows_v.at[1 - buf], sem_g)

            @pl.when(j + 2 < NCHUNK)
            def _():
                pltpu.async_copy(eidx_hbm.at[wid, j + 2],
                                 idx_v.at[lax.rem(j + 2, 4)], sem_i)

        pltpu.make_async_copy(x_hbm.at[idx_v.at[slot, 0]], rows_v.at[buf],
                              sem_g).wait()
        pltpu.sync_copy(rows_v.at[buf], acc_sh.at[idx_v.at[slot, 1]],
                        add=True)
        if with_deg:
            pltpu.sync_copy(ones_v, deg_sh.at[idx_v.at[slot, 1]], add=True)
        return carry

    lax.fori_loop(0, NCHUNK, chunk_body, 0)

    plsc.subcore_barrier()

    pltpu.sync_copy(acc_sh.at[pl.ds(s * RPT, RPT)],
                    part_hbm.at[c, pl.ds(s * RPT, RPT)])
    if with_deg:
        pltpu.sync_copy(deg_sh.at[pl.ds(s * RPT, RPT)],
                        degp_hbm.at[c, pl.ds(s * RPT, RPT)])


def _make_sc_agg(with_deg):
    mesh = plsc.VectorSubcoreMesh(core_axis_name="c", subcore_axis_name="s")
    part_t = jax.ShapeDtypeStruct((NC, ACC_ROWS, D), jnp.float32)
    out_type = [part_t] if with_deg else part_t
    scratch = [
        pltpu.VMEM((4, 2, CHUNK), jnp.int32),
        pltpu.VMEM((2, CHUNK, D), jnp.float32),
    ]
    if with_deg:
        out_type.append(jax.ShapeDtypeStruct((NC, ACC_ROWS), jnp.float32))
        scratch.append(pltpu.VMEM((CHUNK,), jnp.float32))
    scratch.append(pltpu.VMEM_SHARED((ACC_ROWS, D), jnp.float32))
    if with_deg:
        scratch.append(pltpu.VMEM_SHARED((ACC_ROWS,), jnp.float32))
    scratch.append(pltpu.SemaphoreType.DMA)
    scratch.append(pltpu.SemaphoreType.DMA)
    return pl.kernel(
        functools.partial(_sc_agg_body, with_deg),
        out_type=out_type,
        mesh=mesh,
        scratch_types=scratch,
    )


_sc_agg_deg = _make_sc_agg(True)
_sc_agg = _make_sc_agg(False)


def _rowmat(a, b):
    return lax.dot_general(a, b, (((1,), (1,)), ((), ())),
                           preferred_element_type=jnp.float32)


def _tc1_body(p0, p1, dall, x, w_l, b_l, w_r, h_ref):
    d = jnp.sum(dall[...], axis=1, keepdims=True)
    agg = (p0[...][0] + p1[...][0]) / jnp.maximum(d, 1.0)
    z = _rowmat(agg, w_l[...]) + _rowmat(x[...], w_r[...]) + b_l[...]
    h_ref[...] = jnp.maximum(z, 0.0)


def _tc2_body(q0, q1, dall, h1, b3, w_l, b_l, w_r, wfc, bfc,
              out_ref, sums, cnts):
    i = pl.program_id(0)
    d = jnp.sum(dall[...], axis=1, keepdims=True)
    agg = (q0[...][0] + q1[...][0]) / jnp.maximum(d, 1.0)
    z = _rowmat(agg, w_l[...]) + _rowmat(h1[...], w_r[...]) + b_l[...]
    h2 = jnp.maximum(z, 0.0)

    b = b3[...][0, 0, :]
    m = (lax.broadcasted_iota(jnp.int32, (G, BS), 0)
         == b[None, :]).astype(jnp.float32)
    ps = lax.dot_general(m, h2, (((1,), (0,)), ((), ())),
                         precision=lax.Precision.HIGHEST,
                         preferred_element_type=jnp.float32)
    pc = lax.dot_general(m, jnp.ones((BS, D), jnp.float32),
                         (((1,), (0,)), ((), ())),
                         preferred_element_type=jnp.float32)

    @pl.when(i == 0)
    def _():
        sums[...] = ps
        cnts[...] = pc

    @pl.when(i > 0)
    def _():
        sums[...] += ps
        cnts[...] += pc

    @pl.when(i == NB - 1)
    def _():
        pooled = sums[...] / jnp.maximum(cnts[...], 1.0)
        out_ref[...] = jnp.sum(pooled * wfc[...], axis=1) + bfc[0]


_row_spec = pl.BlockSpec((BS, D), lambda i: (i, 0))
_p0_spec = pl.BlockSpec((1, BS, D), lambda i: (0, i, 0))
_p1_spec = pl.BlockSpec((1, BS, D), lambda i: (1, i, 0))
_deg_spec = pl.BlockSpec((BS, NC), lambda i: (i, 0))
_w_spec = pl.BlockSpec((D, D), lambda i: (0, 0))
_b_spec = pl.BlockSpec((D,), lambda i: (0,))

_tc1 = pl.pallas_call(
    _tc1_body,
    grid=(NB,),
    in_specs=[_p0_spec, _p1_spec, _deg_spec, _row_spec,
              _w_spec, _b_spec, _w_spec],
    out_specs=_row_spec,
    out_shape=jax.ShapeDtypeStruct((N, D), jnp.float32),
)

_tc2 = pl.pallas_call(
    _tc2_body,
    grid=(NB,),
    in_specs=[_p0_spec, _p1_spec, _deg_spec, _row_spec,
              pl.BlockSpec((1, 1, BS), lambda i: (i, 0, 0)),
              _w_spec, _b_spec, _w_spec,
              pl.BlockSpec((1, D), lambda i: (0, 0)),
              pl.BlockSpec(memory_space=pltpu.MemorySpace.SMEM)],
    out_specs=pl.BlockSpec((G,), lambda i: (0,)),
    out_shape=jax.ShapeDtypeStruct((G,), jnp.float32),
    scratch_shapes=[pltpu.VMEM((G, D), jnp.float32),
                    pltpu.VMEM((G, D), jnp.float32)],
)


def kernel(x, edge_index, batch, W1_l, b1_l, W1_r, W2_l, b2_l, W2_r,
           W_fc, b_fc):
    src = edge_index[0]
    dst = edge_index[1]
    pad = E_PAD - E
    padi = jnp.arange(pad, dtype=jnp.int32)
    src_p = jnp.concatenate([src, padi % N]).reshape(NW, NCHUNK, 1, CHUNK)
    dst_p = jnp.concatenate([dst, N + padi % (ACC_ROWS - N)]
                            ).reshape(NW, NCHUNK, 1, CHUNK)
    eidx = jnp.concatenate([src_p, dst_p], axis=2)
    zrow = jnp.zeros((RPT, D), jnp.float32)
    zdeg = jnp.zeros((RPT,), jnp.float32)
    ones = jnp.ones((CHUNK,), jnp.float32)

    part1, degp = _sc_agg_deg(x, eidx, zrow, zdeg, ones)
    degT = degp[:, :N].T
    h1 = _tc1(part1, part1, degT, x, W1_l, b1_l, W1_r)
    part2 = _sc_agg(h1, eidx, zrow)
    b3 = batch.reshape(NB, 1, BS)
    out = _tc2(part2, part2, degT, h1, b3,
               W2_l, b2_l, W2_r, W_fc, b_fc)
    return out


# speedup vs baseline: 13.7735x; 1.0614x over previous
import functools

import jax
import jax.numpy as jnp
from jax import lax
from jax.experimental import pallas as pl
from jax.experimental.pallas import tpu as pltpu
from jax.experimental.pallas import tpu_sc as plsc

N = 10000
E = 320000
D = 128
G = 16

NC = 2
NS = 16
NW = NC * NS
CHUNK = 128
EPW = E // NW
NCHUNK = -(-EPW // CHUNK)
EPW_PAD = NCHUNK * CHUNK
E_PAD = EPW_PAD * NW
ACC_ROWS = 10240
RPT = ACC_ROWS // NS
ORT = N // NS
DEG_W = 4

BS = 1000
NB = N // BS


def _sc_agg_body(with_deg, *refs):
    if with_deg:
        (x_hbm, eidx_hbm, zrow_hbm, zdeg_hbm, ones_hbm,
         part_hbm, degp_hbm,
         idx_v, rows_v, ones_v, acc_sh, deg_sh, sem_g, sem_i) = refs
    else:
        (x_hbm, eidx_hbm, zrow_hbm,
         part_hbm,
         idx_v, rows_v, acc_sh, sem_g, sem_i) = refs

    c = lax.axis_index("c")
    s = lax.axis_index("s")
    wid = c * NS + s

    pltpu.sync_copy(zrow_hbm, acc_sh.at[pl.ds(s * RPT, RPT)])
    if with_deg:
        pltpu.sync_copy(zdeg_hbm, deg_sh.at[pl.ds(s * RPT, RPT)])
        pltpu.sync_copy(ones_hbm, ones_v)

    pltpu.async_copy(eidx_hbm.at[wid, 0], idx_v.at[0], sem_i)
    pltpu.async_copy(eidx_hbm.at[wid, 1], idx_v.at[1], sem_i)

    plsc.subcore_barrier()

    pltpu.make_async_copy(eidx_hbm.at[wid, 0], idx_v.at[0], sem_i).wait()
    pltpu.async_copy(x_hbm.at[idx_v.at[0, 0]], rows_v.at[0], sem_g)

    def chunk_body(j, carry):
        buf = lax.rem(j, 2)
        slot = lax.rem(j, 4)

        @pl.when(j + 1 < NCHUNK)
        def _():
            nslot = lax.rem(j + 1, 4)
            pltpu.make_async_copy(eidx_hbm.at[wid, j + 1], idx_v.at[nslot],
                                  sem_i).wait()
            pltpu.async_copy(x_hbm.at[idx_v.at[nslot, 0]],
                             rows_v.at[1 - buf], sem_g)

            @pl.when(j + 2 < NCHUNK)
            def _():
                pltpu.async_copy(eidx_hbm.at[wid, j + 2],
                                 idx_v.at[lax.rem(j + 2, 4)], sem_i)

        pltpu.make_async_copy(x_hbm.at[idx_v.at[slot, 0]], rows_v.at[buf],
                              sem_g).wait()
        pltpu.sync_copy(rows_v.at[buf], acc_sh.at[idx_v.at[slot, 1]],
                        add=True)
        if with_deg:
            pltpu.sync_copy(ones_v, deg_sh.at[idx_v.at[slot, 1]], add=True)
        return carry

    lax.fori_loop(0, NCHUNK, chunk_body, 0)

    plsc.subcore_barrier()

    pltpu.sync_copy(acc_sh.at[pl.ds(s * RPT, RPT)],
                    part_hbm.at[c, pl.ds(s * RPT, RPT)])
    if with_deg:
        pltpu.sync_copy(deg_sh.at[pl.ds(s * RPT, RPT)],
                        degp_hbm.at[c, pl.ds(s * RPT, RPT)])


def _make_sc_agg(with_deg):
    mesh = plsc.VectorSubcoreMesh(core_axis_name="c", subcore_axis_name="s")
    part_t = jax.ShapeDtypeStruct((NC, ACC_ROWS, D), jnp.float32)
    out_type = [part_t] if with_deg else part_t
    scratch = [
        pltpu.VMEM((4, 2, CHUNK), jnp.int32),
        pltpu.VMEM((2, CHUNK, D), jnp.float32),
    ]
    if with_deg:
        out_type.append(jax.ShapeDtypeStruct((NC, ACC_ROWS), jnp.float32))
        scratch.append(pltpu.VMEM((CHUNK,), jnp.float32))
    scratch.append(pltpu.VMEM_SHARED((ACC_ROWS, D), jnp.float32))
    if with_deg:
        scratch.append(pltpu.VMEM_SHARED((ACC_ROWS,), jnp.float32))
    scratch.append(pltpu.SemaphoreType.DMA)
    scratch.append(pltpu.SemaphoreType.DMA)
    return pl.kernel(
        functools.partial(_sc_agg_body, with_deg),
        out_type=out_type,
        mesh=mesh,
        scratch_types=scratch,
    )


_sc_agg_deg = _make_sc_agg(True)
_sc_agg = _make_sc_agg(False)


def _rowmat(a, b):
    return lax.dot_general(a, b, (((1,), (1,)), ((), ())),
                           preferred_element_type=jnp.float32)


def _tc1_body(p0, p1, dall, x, w_l, b_l, w_r, h_ref):
    d = jnp.sum(dall[...], axis=1, keepdims=True)
    agg = (p0[...][0] + p1[...][0]) / jnp.maximum(d, 1.0)
    z = _rowmat(agg, w_l[...]) + _rowmat(x[...], w_r[...]) + b_l[...]
    h_ref[...] = jnp.maximum(z, 0.0)


def _tc2_body(q0, q1, dall, h1, b3, w_l, b_l, w_r, wfc, bfc,
              out_ref, sums, cnts):
    i = pl.program_id(0)
    d = jnp.sum(dall[...], axis=1, keepdims=True)
    agg = (q0[...][0] + q1[...][0]) / jnp.maximum(d, 1.0)
    z = _rowmat(agg, w_l[...]) + _rowmat(h1[...], w_r[...]) + b_l[...]
    h2 = jnp.maximum(z, 0.0)

    b = b3[...][0, 0, :]
    m = (lax.broadcasted_iota(jnp.int32, (G, BS), 0)
         == b[None, :]).astype(jnp.float32)
    ps = lax.dot_general(m, h2, (((1,), (0,)), ((), ())),
                         precision=lax.Precision.HIGHEST,
                         preferred_element_type=jnp.float32)
    pc = lax.dot_general(m, jnp.ones((BS, D), jnp.float32),
                         (((1,), (0,)), ((), ())),
                         preferred_element_type=jnp.float32)

    @pl.when(i == 0)
    def _():
        sums[...] = ps
        cnts[...] = pc

    @pl.when(i > 0)
    def _():
        sums[...] += ps
        cnts[...] += pc

    @pl.when(i == NB - 1)
    def _():
        pooled = sums[...] / jnp.maximum(cnts[...], 1.0)
        out_ref[...] = jnp.sum(pooled * wfc[...], axis=1) + bfc[0]


_row_spec = pl.BlockSpec((BS, D), lambda i: (i, 0))
_p0_spec = pl.BlockSpec((1, BS, D), lambda i: (0, i, 0))
_p1_spec = pl.BlockSpec((1, BS, D), lambda i: (1, i, 0))
_deg_spec = pl.BlockSpec((BS, NC), lambda i: (i, 0))
_w_spec = pl.BlockSpec((D, D), lambda i: (0, 0))
_b_spec = pl.BlockSpec((D,), lambda i: (0,))

_tc1 = pl.pallas_call(
    _tc1_body,
    grid=(NB,),
    in_specs=[_p0_spec, _p1_spec, _deg_spec, _row_spec,
              _w_spec, _b_spec, _w_spec],
    out_specs=_row_spec,
    out_shape=jax.ShapeDtypeStruct((N, D), jnp.float32),
)

_tc2 = pl.pallas_call(
    _tc2_body,
    grid=(NB,),
    in_specs=[_p0_spec, _p1_spec, _deg_spec, _row_spec,
              pl.BlockSpec((1, 1, BS), lambda i: (i, 0, 0)),
              _w_spec, _b_spec, _w_spec,
              pl.BlockSpec((1, D), lambda i: (0, 0)),
              pl.BlockSpec(memory_space=pltpu.MemorySpace.SMEM)],
    out_specs=pl.BlockSpec((G,), lambda i: (0,)),
    out_shape=jax.ShapeDtypeStruct((G,), jnp.float32),
    scratch_shapes=[pltpu.VMEM((G, D), jnp.float32),
                    pltpu.VMEM((G, D), jnp.float32)],
)


def kernel(x, edge_index, batch, W1_l, b1_l, W1_r, W2_l, b2_l, W2_r,
           W_fc, b_fc):
    src = edge_index[0]
    dst = edge_index[1]
    pad = E_PAD - E
    padi = jnp.arange(pad, dtype=jnp.int32)
    src_p = jnp.concatenate([src, padi % N]).reshape(NW, NCHUNK, 1, CHUNK)
    dst_p = jnp.concatenate([dst, N + padi % (ACC_ROWS - N)]
                            ).reshape(NW, NCHUNK, 1, CHUNK)
    eidx = jnp.concatenate([src_p, dst_p], axis=2)
    zrow = jnp.zeros((RPT, D), jnp.float32)
    zdeg = jnp.zeros((RPT,), jnp.float32)
    ones = jnp.ones((CHUNK,), jnp.float32)

    part1, degp = _sc_agg_deg(x, eidx, zrow, zdeg, ones)
    degT = degp[:, :N].T
    h1 = _tc1(part1, part1, degT, x, W1_l, b1_l, W1_r)
    part2 = _sc_agg(h1, eidx, zrow)
    b3 = batch.reshape(NB, 1, BS)
    out = _tc2(part2, part2, degT, h1, b3,
               W2_l, b2_l, W2_r, W_fc, b_fc)
    return out


# speedup vs baseline: 13.9779x; 1.0148x over previous
import functools

import jax
import jax.numpy as jnp
from jax import lax
from jax.experimental import pallas as pl
from jax.experimental.pallas import tpu as pltpu
from jax.experimental.pallas import tpu_sc as plsc

N = 10000
E = 320000
D = 128
G = 16

NC = 2
NS = 16
NW = NC * NS
CHUNK = 128
EPW = E // NW
NCHUNK = -(-EPW // CHUNK)
EPW_PAD = NCHUNK * CHUNK
E_PAD = EPW_PAD * NW
ACC_ROWS = 10240
RPT = ACC_ROWS // NS
ORT = N // NS
DEG_W = 4

BS = 2000
NB = N // BS


def _sc_agg_body(with_deg, *refs):
    if with_deg:
        (x_hbm, eidx_hbm, zrow_hbm, zdeg_hbm, ones_hbm,
         part_hbm, degp_hbm,
         idx_v, rows_v, ones_v, acc_sh, deg_sh, sem_g, sem_i) = refs
    else:
        (x_hbm, eidx_hbm, zrow_hbm,
         part_hbm,
         idx_v, rows_v, acc_sh, sem_g, sem_i) = refs

    c = lax.axis_index("c")
    s = lax.axis_index("s")
    wid = c * NS + s

    pltpu.sync_copy(zrow_hbm, acc_sh.at[pl.ds(s * RPT, RPT)])
    if with_deg:
        pltpu.sync_copy(zdeg_hbm, deg_sh.at[pl.ds(s * RPT, RPT)])
        pltpu.sync_copy(ones_hbm, ones_v)

    pltpu.async_copy(eidx_hbm.at[wid, 0], idx_v.at[0], sem_i)
    pltpu.async_copy(eidx_hbm.at[wid, 1], idx_v.at[1], sem_i)

    plsc.subcore_barrier()

    pltpu.make_async_copy(eidx_hbm.at[wid, 0], idx_v.at[0], sem_i).wait()
    pltpu.async_copy(x_hbm.at[idx_v.at[0, 0]], rows_v.at[0], sem_g)

    def chunk_body(j, carry):
        buf = lax.rem(j, 2)
        slot = lax.rem(j, 4)

        @pl.when(j + 1 < NCHUNK)
        def _():
            nslot = lax.rem(j + 1, 4)
            pltpu.make_async_copy(eidx_hbm.at[wid, j + 1], idx_v.at[nslot],
                                  sem_i).wait()
            pltpu.async_copy(x_hbm.at[idx_v.at[nslot, 0]],
                             rows_v.at[1 - buf], sem_g)

            @pl.when(j + 2 < NCHUNK)
            def _():
                pltpu.async_copy(eidx_hbm.at[wid, j + 2],
                                 idx_v.at[lax.rem(j + 2, 4)], sem_i)

        pltpu.make_async_copy(x_hbm.at[idx_v.at[slot, 0]], rows_v.at[buf],
                              sem_g).wait()
        pltpu.sync_copy(rows_v.at[buf], acc_sh.at[idx_v.at[slot, 1]],
                        add=True)
        if with_deg:
            pltpu.sync_copy(ones_v, deg_sh.at[idx_v.at[slot, 1]], add=True)
        return carry

    lax.fori_loop(0, NCHUNK, chunk_body, 0)

    plsc.subcore_barrier()

    pltpu.sync_copy(acc_sh.at[pl.ds(s * RPT, RPT)],
                    part_hbm.at[c, pl.ds(s * RPT, RPT)])
    if with_deg:
        pltpu.sync_copy(deg_sh.at[pl.ds(s * RPT, RPT)],
                        degp_hbm.at[c, pl.ds(s * RPT, RPT)])


def _make_sc_agg(with_deg):
    mesh = plsc.VectorSubcoreMesh(core_axis_name="c", subcore_axis_name="s")
    part_t = jax.ShapeDtypeStruct((NC, ACC_ROWS, D), jnp.float32)
    out_type = [part_t] if with_deg else part_t
    scratch = [
        pltpu.VMEM((4, 2, CHUNK), jnp.int32),
        pltpu.VMEM((2, CHUNK, D), jnp.float32),
    ]
    if with_deg:
        out_type.append(jax.ShapeDtypeStruct((NC, ACC_ROWS), jnp.float32))
        scratch.append(pltpu.VMEM((CHUNK,), jnp.float32))
    scratch.append(pltpu.VMEM_SHARED((ACC_ROWS, D), jnp.float32))
    if with_deg:
        scratch.append(pltpu.VMEM_SHARED((ACC_ROWS,), jnp.float32))
    scratch.append(pltpu.SemaphoreType.DMA)
    scratch.append(pltpu.SemaphoreType.DMA)
    return pl.kernel(
        functools.partial(_sc_agg_body, with_deg),
        out_type=out_type,
        mesh=mesh,
        scratch_types=scratch,
    )


_sc_agg_deg = _make_sc_agg(True)
_sc_agg = _make_sc_agg(False)


def _rowmat(a, b):
    return lax.dot_general(a, b, (((1,), (1,)), ((), ())),
                           preferred_element_type=jnp.float32)


def _tc1_body(p0, p1, dall, x, w_l, b_l, w_r, h_ref):
    d = jnp.sum(dall[...], axis=1, keepdims=True)
    agg = (p0[...][0] + p1[...][0]) / jnp.maximum(d, 1.0)
    z = _rowmat(agg, w_l[...]) + _rowmat(x[...], w_r[...]) + b_l[...]
    h_ref[...] = jnp.maximum(z, 0.0)


def _tc2_body(q0, q1, dall, h1, b3, w_l, b_l, w_r, wfc, bfc,
              out_ref, sums, cnts):
    i = pl.program_id(0)
    d = jnp.sum(dall[...], axis=1, keepdims=True)
    agg = (q0[...][0] + q1[...][0]) / jnp.maximum(d, 1.0)
    z = _rowmat(agg, w_l[...]) + _rowmat(h1[...], w_r[...]) + b_l[...]
    h2 = jnp.maximum(z, 0.0)

    b = b3[...][0, 0, :]
    m = (lax.broadcasted_iota(jnp.int32, (G, BS), 0)
         == b[None, :]).astype(jnp.float32)
    ps = lax.dot_general(m, h2, (((1,), (0,)), ((), ())),
                         precision=lax.Precision.HIGHEST,
                         preferred_element_type=jnp.float32)
    pc = lax.dot_general(m, jnp.ones((BS, D), jnp.float32),
                         (((1,), (0,)), ((), ())),
                         preferred_element_type=jnp.float32)

    @pl.when(i == 0)
    def _():
        sums[...] = ps
        cnts[...] = pc

    @pl.when(i > 0)
    def _():
        sums[...] += ps
        cnts[...] += pc

    @pl.when(i == NB - 1)
    def _():
        pooled = sums[...] / jnp.maximum(cnts[...], 1.0)
        out_ref[...] = jnp.sum(pooled * wfc[...], axis=1) + bfc[0]


_row_spec = pl.BlockSpec((BS, D), lambda i: (i, 0))
_p0_spec = pl.BlockSpec((1, BS, D), lambda i: (0, i, 0))
_p1_spec = pl.BlockSpec((1, BS, D), lambda i: (1, i, 0))
_deg_spec = pl.BlockSpec((BS, NC), lambda i: (i, 0))
_w_spec = pl.BlockSpec((D, D), lambda i: (0, 0))
_b_spec = pl.BlockSpec((D,), lambda i: (0,))

_tc1 = pl.pallas_call(
    _tc1_body,
    grid=(NB,),
    in_specs=[_p0_spec, _p1_spec, _deg_spec, _row_spec,
              _w_spec, _b_spec, _w_spec],
    out_specs=_row_spec,
    out_shape=jax.ShapeDtypeStruct((N, D), jnp.float32),
)

_tc2 = pl.pallas_call(
    _tc2_body,
    grid=(NB,),
    in_specs=[_p0_spec, _p1_spec, _deg_spec, _row_spec,
              pl.BlockSpec((1, 1, BS), lambda i: (i, 0, 0)),
              _w_spec, _b_spec, _w_spec,
              pl.BlockSpec((1, D), lambda i: (0, 0)),
              pl.BlockSpec(memory_space=pltpu.MemorySpace.SMEM)],
    out_specs=pl.BlockSpec((G,), lambda i: (0,)),
    out_shape=jax.ShapeDtypeStruct((G,), jnp.float32),
    scratch_shapes=[pltpu.VMEM((G, D), jnp.float32),
                    pltpu.VMEM((G, D), jnp.float32)],
)


def kernel(x, edge_index, batch, W1_l, b1_l, W1_r, W2_l, b2_l, W2_r,
           W_fc, b_fc):
    src = edge_index[0]
    dst = edge_index[1]
    pad = E_PAD - E
    padi = jnp.arange(pad, dtype=jnp.int32)
    src_p = jnp.concatenate([src, padi % N]).reshape(NW, NCHUNK, 1, CHUNK)
    dst_p = jnp.concatenate([dst, N + padi % (ACC_ROWS - N)]
                            ).reshape(NW, NCHUNK, 1, CHUNK)
    eidx = jnp.concatenate([src_p, dst_p], axis=2)
    zrow = jnp.zeros((RPT, D), jnp.float32)
    zdeg = jnp.zeros((RPT,), jnp.float32)
    ones = jnp.ones((CHUNK,), jnp.float32)

    part1, degp = _sc_agg_deg(x, eidx, zrow, zdeg, ones)
    degT = degp[:, :N].T
    h1 = _tc1(part1, part1, degT, x, W1_l, b1_l, W1_r)
    part2 = _sc_agg(h1, eidx, zrow)
    b3 = batch.reshape(NB, 1, BS)
    out = _tc2(part2, part2, degT, h1, b3,
               W2_l, b2_l, W2_r, W_fc, b_fc)
    return out
